# baseline (speedup 1.0000x reference)
import functools

import jax
import jax.numpy as jnp
from jax import lax
from jax.experimental import pallas as pl
from jax.experimental.pallas import tpu as pltpu
from jax.experimental.pallas import tpu_sc as plsc

VOCAB = 100000
DIM = 300
BATCH = 1024
SEQ = 200

_NW = 32
_NBANDS = SEQ // 8
_DPW = DIM // _NW
_EXTRA = DIM - _DPW * _NW


def _make_gather():
    mesh = plsc.VectorSubcoreMesh(core_axis_name="c", subcore_axis_name="s")

    @functools.partial(
        pl.kernel,
        mesh=mesh,
        compiler_params=pltpu.CompilerParams(needs_layout_passes=False),
        out_type=jax.ShapeDtypeStruct((DIM, SEQ, BATCH), jnp.float32),
        scratch_types=[
            pltpu.VMEM((VOCAB,), jnp.float32),
            pltpu.VMEM((8, BATCH), jnp.int32),
            pltpu.VMEM((8, BATCH), jnp.int32),
            pltpu.VMEM((8, BATCH // 2), jnp.float32),
            pltpu.VMEM((8, BATCH // 2), jnp.float32),
            pltpu.SemaphoreType.DMA,
            pltpu.SemaphoreType.DMA,
            pltpu.SemaphoreType.DMA,
            pltpu.SemaphoreType.DMA,
        ],
    )
    def gather_kernel(ids_hbm, table_hbm, out_hbm, row_v, ids0_v, ids1_v,
                      outa_v, outb_v, isem0, isem1, osem_a, osem_b):
        sid = lax.axis_index("s")
        cid = lax.axis_index("c")
        wid = sid * 2 + cid

        ids_bufs = ((ids0_v, isem0), (ids1_v, isem1))

        def ids_copy(band, i):
            buf, sem = ids_bufs[i % 2]
            return pltpu.make_async_copy(
                ids_hbm.at[pl.ds(band * 8, 8)], buf, sem)

        halves = (
            (outa_v, osem_a, 0),
            (outb_v, osem_b, BATCH // 2),
        )

        def out_copy(d, band, half):
            dst, sem, off = halves[half]
            return pltpu.make_async_copy(
                dst, out_hbm.at[d, pl.ds(band * 8, 8), pl.ds(off, BATCH // 2)],
                sem)

        def gather_half(buf, half):
            dst, _, off = halves[half]

            @plsc.parallel_loop(0, BATCH // 2, step=16, unroll=3)
            def gather_body(c):
                for r in range(8):
                    iv = buf[r, pl.ds(c + off, 16)]
                    dst[r, pl.ds(c, 16)] = plsc.load_gather(row_v, [iv])

        def gather_band(d, band, buf, first_band):
            for half in (0, 1):
                if not first_band:
                    out_copy(d, band - 1, half).wait()
                gather_half(buf, half)
                out_copy(d, band, half).start()

        def emit_plane(d, b0, n):
            npairs = (n - 1) // 2
            trailing = (n - 1) - 2 * npairs

            ids_copy(b0, 0).wait()
            gather_band(d, b0, ids0_v, True)
            if n > 1:
                ids_copy(b0 + 1, 1).start()

            def pair_body(p, carry):
                band_a = b0 + 1 + 2 * p
                ids_copy(band_a + 1, 0).start()
                ids_copy(band_a, 1).wait()
                gather_band(d, band_a, ids1_v, False)

                band_b = band_a + 1

                @pl.when(2 * p + 3 <= n - 1)
                def _():
                    ids_copy(band_b + 1, 1).start()

                ids_copy(band_b, 0).wait()
                gather_band(d, band_b, ids0_v, False)
                return carry

            if npairs > 0:
                lax.fori_loop(0, npairs, pair_body, 0)
            if trailing:
                band_t = b0 + n - 1
                ids_copy(band_t, 1).wait()
                gather_band(d, band_t, ids1_v, False)
            for half in (0, 1):
                out_copy(d, b0 + n - 1, half).wait()

        ids_copy(0, 0).start()

        def plane_body(k, carry):
            d = _DPW * wid + k
            pltpu.sync_copy(table_hbm.at[d], row_v)
            emit_plane(d, 0, _NBANDS)

            @pl.when(k + 1 < _DPW)
            def _():
                ids_copy(0, 0).start()

            return carry

        lax.fori_loop(0, _DPW, plane_body, 0)

        @pl.when(sid < _EXTRA)
        def _():
            d = _DPW * _NW + sid

            @pl.when(cid == 0)
            def _():
                ids_copy(0, 0).start()
                pltpu.sync_copy(table_hbm.at[d], row_v)
                emit_plane(d, 0, 13)

            @pl.when(cid == 1)
            def _():
                ids_copy(13, 0).start()
                pltpu.sync_copy(table_hbm.at[d], row_v)
                emit_plane(d, 13, _NBANDS - 13)

    return gather_kernel


_gather = _make_gather()


def kernel(word_ids, emb_weight):
    out_p = _gather(word_ids.T, emb_weight.T)
    return out_p.transpose(2, 1, 0)


# speedup vs baseline: 1.2619x; 1.2619x over previous
import functools

import jax
import jax.numpy as jnp
from jax import lax
from jax.experimental import pallas as pl
from jax.experimental.pallas import tpu as pltpu
from jax.experimental.pallas import tpu_sc as plsc

VOCAB = 100000
DIM = 300
BATCH = 1024
SEQ = 200

_NW = 32
_NBANDS = SEQ // 8
_DPW = DIM // _NW
_EXTRA = DIM - _DPW * _NW


def _make_gather():
    mesh = plsc.VectorSubcoreMesh(core_axis_name="c", subcore_axis_name="s")

    @functools.partial(
        pl.kernel,
        mesh=mesh,
        compiler_params=pltpu.CompilerParams(needs_layout_passes=False),
        out_type=jax.ShapeDtypeStruct((DIM, SEQ, BATCH), jnp.float32),
        scratch_types=[
            pltpu.VMEM((VOCAB,), jnp.float32),
            pltpu.VMEM((8, BATCH), jnp.int32),
            pltpu.VMEM((8, BATCH), jnp.int32),
            pltpu.VMEM((8, BATCH // 2), jnp.float32),
            pltpu.VMEM((8, BATCH // 2), jnp.float32),
            pltpu.SemaphoreType.DMA,
            pltpu.SemaphoreType.DMA,
            pltpu.SemaphoreType.DMA,
            pltpu.SemaphoreType.DMA,
        ],
    )
    def gather_kernel(ids_hbm, table_hbm, out_hbm, row_v, ids0_v, ids1_v,
                      outa_v, outb_v, isem0, isem1, osem_a, osem_b):
        sid = lax.axis_index("s")
        cid = lax.axis_index("c")
        wid = sid * 2 + cid

        ids_bufs = ((ids0_v, isem0), (ids1_v, isem1))

        def ids_copy(band, i):
            buf, sem = ids_bufs[i % 2]
            return pltpu.make_async_copy(
                ids_hbm.at[pl.ds(band * 8, 8)], buf, sem)

        halves = (
            (outa_v, osem_a, 0),
            (outb_v, osem_b, BATCH // 2),
        )

        def out_copy(d, band, half):
            dst, sem, off = halves[half]
            return pltpu.make_async_copy(
                dst, out_hbm.at[d, pl.ds(band * 8, 8), pl.ds(off, BATCH // 2)],
                sem)

        def gather_half(buf, half):
            dst, _, off = halves[half]

            @plsc.parallel_loop(0, BATCH // 2, step=16, unroll=2)
            def gather_body(c):
                for r in range(8):
                    iv = buf[r, pl.ds(c + off, 16)]
                    dst[r, pl.ds(c, 16)] = plsc.load_gather(row_v, [iv])

        def gather_band(d, band, buf, first_band):
            for half in (0, 1):
                if not first_band:
                    out_copy(d, band - 1, half).wait()
                gather_half(buf, half)
                out_copy(d, band, half).start()

        def emit_plane(d, b0, n):
            npairs = (n - 1) // 2
            trailing = (n - 1) - 2 * npairs

            ids_copy(b0, 0).wait()
            gather_band(d, b0, ids0_v, True)
            if n > 1:
                ids_copy(b0 + 1, 1).start()

            def pair_body(p, carry):
                band_a = b0 + 1 + 2 * p
                ids_copy(band_a + 1, 0).start()
                ids_copy(band_a, 1).wait()
                gather_band(d, band_a, ids1_v, False)

                band_b = band_a + 1

                @pl.when(2 * p + 3 <= n - 1)
                def _():
                    ids_copy(band_b + 1, 1).start()

                ids_copy(band_b, 0).wait()
                gather_band(d, band_b, ids0_v, False)
                return carry

            if npairs > 0:
                lax.fori_loop(0, npairs, pair_body, 0)
            if trailing:
                band_t = b0 + n - 1
                ids_copy(band_t, 1).wait()
                gather_band(d, band_t, ids1_v, False)
            for half in (0, 1):
                out_copy(d, b0 + n - 1, half).wait()

        ids_copy(0, 0).start()

        def plane_body(k, carry):
            d = _DPW * wid + k
            pltpu.sync_copy(table_hbm.at[d], row_v)
            emit_plane(d, 0, _NBANDS)

            @pl.when(k + 1 < _DPW)
            def _():
                ids_copy(0, 0).start()

            return carry

        lax.fori_loop(0, _DPW, plane_body, 0)

        @pl.when(sid < _EXTRA)
        def _():
            d = _DPW * _NW + sid

            @pl.when(cid == 0)
            def _():
                ids_copy(0, 0).start()
                pltpu.sync_copy(table_hbm.at[d], row_v)
                emit_plane(d, 0, 13)

            @pl.when(cid == 1)
            def _():
                ids_copy(13, 0).start()
                pltpu.sync_copy(table_hbm.at[d], row_v)
                emit_plane(d, 13, _NBANDS - 13)

    return gather_kernel


_gather = _make_gather()


def kernel(word_ids, emb_weight):
    out_p = _gather(word_ids.T, emb_weight.T)
    return out_p.transpose(2, 1, 0)
